# Initial kernel scaffold; baseline (speedup 1.0000x reference)
import jax
import jax.numpy as jnp
from jax.experimental import pallas as pl


def kernel(x, conv_w, conv_b, gamma, beta):
    raise NotImplementedError("write your pallas kernel here")


# speedup vs baseline: 1.1031x; 1.1031x over previous
import jax
import jax.numpy as jnp
from jax.experimental import pallas as pl
from jax.experimental.pallas import tpu as pltpu

_BN_EPS = 4e-5


def _make_body(n_imgs, m_total, bn_eps):
    def _body(x_ref, w_ref, g_ref, b_ref, o_ref,
              y_buf, sum_ref, sq_ref, scale_ref, shift_ref):
        p = pl.program_id(0)
        n = pl.program_id(1)

        @pl.when(p == 0)
        def _compute_and_stats():
            @pl.when(n == 0)
            def _init():
                sum_ref[...] = jnp.zeros_like(sum_ref)
                sq_ref[...] = jnp.zeros_like(sq_ref)

            y = jnp.dot(w_ref[...], x_ref[0],
                        preferred_element_type=jnp.float32)
            sum_ref[...] += jnp.sum(y, axis=1, keepdims=True)
            sq_ref[...] += jnp.sum(y * y, axis=1, keepdims=True)
            y_buf[n] = y.astype(jnp.bfloat16)

        @pl.when(p == 1)
        def _normalize():
            @pl.when(n == 0)
            def _fold_stats():
                mean = sum_ref[...] / m_total
                var = jnp.maximum(sq_ref[...] / m_total - mean * mean, 0.0)
                inv_std = 1.0 / jnp.sqrt(var + bn_eps)
                scale = g_ref[...] * inv_std
                scale_ref[...] = scale
                shift_ref[...] = b_ref[...] - mean * scale

            y = y_buf[n].astype(jnp.float32)
            o_ref[0] = jnp.maximum(y * scale_ref[...] + shift_ref[...],
                                   0.0).astype(o_ref.dtype)

    return _body


def kernel(x, conv_w, conv_b, gamma, beta):
    N, Cin, H, W = x.shape
    Cout = conv_w.shape[0]
    HW = H * W
    M = N * HW
    del conv_b

    x3 = x.reshape(N, Cin, HW)
    w_mat = conv_w.reshape(Cout, Cin).astype(jnp.float32)
    g2 = gamma.astype(jnp.float32).reshape(Cout, 1)
    b2 = beta.astype(jnp.float32).reshape(Cout, 1)

    x_spec = pl.BlockSpec(
        (1, Cin, HW), lambda p, n: (jnp.where(p == 0, n, N - 1), 0, 0))
    o_spec = pl.BlockSpec(
        (1, Cout, HW), lambda p, n: (jnp.where(p == 0, 0, n), 0, 0))
    const_spec = pl.BlockSpec((Cout, Cin), lambda p, n: (0, 0))
    vec_spec = pl.BlockSpec((Cout, 1), lambda p, n: (0, 0))

    cost = pl.CostEstimate(
        flops=2 * M * Cin * Cout + 7 * M * Cout,
        transcendentals=Cout,
        bytes_accessed=M * Cin * 4 + M * Cout * 4 + Cout * Cin * 4)

    out3 = pl.pallas_call(
        _make_body(N, M, _BN_EPS),
        out_shape=jax.ShapeDtypeStruct((N, Cout, HW), x.dtype),
        grid=(2, N),
        in_specs=[x_spec, const_spec, vec_spec, vec_spec],
        out_specs=o_spec,
        scratch_shapes=[
            pltpu.VMEM((N, Cout, HW), jnp.bfloat16),
            pltpu.VMEM((Cout, 1), jnp.float32),
            pltpu.VMEM((Cout, 1), jnp.float32),
            pltpu.VMEM((Cout, 1), jnp.float32),
            pltpu.VMEM((Cout, 1), jnp.float32),
        ],
        compiler_params=pltpu.CompilerParams(
            dimension_semantics=("arbitrary", "arbitrary"),
            vmem_limit_bytes=60 * 1024 * 1024),
        cost_estimate=cost,
    )(x3, w_mat, g2, b2)

    return out3.reshape(N, Cout, H, W)
